# baseline (speedup 1.0000x reference)
import jax
import jax.numpy as jnp
from jax import lax
from jax.experimental import pallas as pl
from jax.experimental.pallas import tpu as pltpu
from jax.experimental.pallas import tpu_sc as plsc

N = 10000
NP = 10240
E = 160000
EP = 163840
CIN = 32
B = 8
NW = 32
NCH = 40
CH = 128
NC = 2
NS = 16
RPW = NP // NS

_f32 = jnp.float32


NBUF = 4


def _sc_pass_body(table, gidx, sidx, zrows, out, gidx_v, sidx_v, rows_v,
                  buf_v, acc, sem0, sem1, sem2, sem3):
    c = lax.axis_index("c")
    s = lax.axis_index("s")
    wid = s * NC + c
    base = s * RPW
    sems = (sem0, sem1, sem2, sem3)
    pltpu.sync_copy(gidx.at[wid], gidx_v)
    for k in range(NBUF):
        pltpu.async_copy(table.at[gidx_v.at[k]], rows_v.at[k], sems[k])
    pltpu.sync_copy(sidx.at[wid], sidx_v)
    pltpu.sync_copy(zrows.at[pl.ds(base, RPW)], buf_v)
    pltpu.sync_copy(buf_v, acc.at[pl.ds(base, RPW)])
    plsc.subcore_barrier()

    def grp(g, carry):
        j0 = g * NBUF
        jn = j0 + NBUF
        for k in range(NBUF):
            pltpu.make_async_copy(table.at[gidx_v.at[j0 + k]],
                                  rows_v.at[k], sems[k]).wait()
            pltpu.sync_copy(rows_v.at[k], acc.at[sidx_v.at[j0 + k]],
                            add=True)

            @pl.when(jn + k < NCH)
            def _():
                pltpu.async_copy(table.at[gidx_v.at[jn + k]], rows_v.at[k],
                                 sems[k])
        return carry

    lax.fori_loop(0, NCH // NBUF, grp, 0)
    plsc.subcore_barrier()
    pltpu.sync_copy(acc.at[pl.ds(base, RPW)], out.at[c, pl.ds(base, RPW)])


NPT = EP // NW // 16


def _sc_deg1_body(srcw, dstw, x0f, zrows, tpo, tabx1, cso, cdo,
                  sidx_v, didx_v, gidx_v, six_v, rows_v, buf_v, x0_v, tb_v,
                  hist_s, hist_d, red_v, cs_v, cd_v,
                  acc, hsh_s, hsh_d, sem0, sem1, sem2, sem3):
    c = lax.axis_index("c")
    s = lax.axis_index("s")
    wid = s * NC + c
    base = s * RPW
    cnp = c * NP
    sems = (sem0, sem1, sem2, sem3)
    iota = lax.iota(jnp.int32, 16)
    rep = iota // B
    lane8 = iota % B

    pltpu.sync_copy(srcw.at[pl.ds(2 * s, 2)], sidx_v)
    pltpu.sync_copy(dstw.at[pl.ds(2 * s, 2)], didx_v)
    pltpu.sync_copy(srcw.at[wid], gidx_v)
    pltpu.sync_copy(dstw.at[wid], six_v)
    pltpu.sync_copy(zrows.at[pl.ds(base, RPW)], buf_v)
    pltpu.sync_copy(buf_v, acc.at[pl.ds(base, RPW)])
    pltpu.sync_copy(x0f.at[pl.ds(base * B, RPW * B)], x0_v)

    def zbody(i, carry):
        hist_s[pl.ds(i * 16, 16)] = jnp.zeros((16,), _f32)
        hist_d[pl.ds(i * 16, 16)] = jnp.zeros((16,), _f32)
        return carry

    lax.fori_loop(0, NP // 16, zbody, 0)
    ones = jnp.full((16,), 1.0, _f32)

    def hbody(r, carry):
        for j in range(2):
            for k in range(CH // 16):
                plsc.addupdate_scatter(
                    hist_s, [sidx_v[j, r, pl.ds(k * 16, 16)]], ones)
                plsc.addupdate_scatter(
                    hist_d, [didx_v[j, r, pl.ds(k * 16, 16)]], ones)
        return carry

    lax.fori_loop(0, NCH, hbody, 0)
    pltpu.sync_copy(hist_s, hsh_s.at[s])
    pltpu.sync_copy(hist_d, hsh_d.at[s])
    plsc.subcore_barrier()

    def _rsqrt_reduce(hsh, dst_v):
        for r in range(NS):
            pltpu.sync_copy(hsh.at[r, pl.ds(base, RPW)], red_v.at[r])

        def cbody(i, carry):
            tot = red_v[0, pl.ds(i * 16, 16)]
            for r in range(1, NS):
                tot = tot + red_v[r, pl.ds(i * 16, 16)]
            x = jnp.maximum(tot, 1.0)
            yi = 0x5F3759DF - lax.shift_right_logical(
                plsc.bitcast(x, jnp.int32), 1)
            y = plsc.bitcast(yi, _f32)
            for _ in range(3):
                y = y * (1.5 - 0.5 * x * y * y)
            dst_v[pl.ds(i * 16, 16)] = y
            return carry

        lax.fori_loop(0, RPW // 16, cbody, 0)

    _rsqrt_reduce(hsh_s, cs_v)
    _rsqrt_reduce(hsh_d, cd_v)

    @pl.when(c == 0)
    def _():
        pltpu.sync_copy(cs_v, cso.at[pl.ds(base, RPW)])
        pltpu.sync_copy(cd_v, cdo.at[pl.ds(base, RPW)])

    def tbody(i, carry):
        nidx = rep + 2 * i
        csb = plsc.load_gather(cs_v, [nidx])
        plsc.store_scatter(tb_v, [nidx, lane8],
                           x0_v[pl.ds(i * 16, 16)] * csb)
        return carry

    lax.fori_loop(0, RPW * B // 16, tbody, 0)
    pltpu.sync_copy(tb_v, tabx1.at[pl.ds(cnp + base, RPW)])

    def obody(j, carry):
        for k in range(CH // 16):
            gidx_v[j, pl.ds(k * 16, 16)] = (
                gidx_v[j, pl.ds(k * 16, 16)] + cnp)
        return carry

    lax.fori_loop(0, NCH, obody, 0)
    plsc.subcore_barrier()

    for k in range(NBUF):
        pltpu.async_copy(tabx1.at[gidx_v.at[k]], rows_v.at[k], sems[k])

    def grp(g, carry):
        j0 = g * NBUF
        jn = j0 + NBUF
        for k in range(NBUF):
            pltpu.make_async_copy(tabx1.at[gidx_v.at[j0 + k]],
                                  rows_v.at[k], sems[k]).wait()
            pltpu.sync_copy(rows_v.at[k], acc.at[six_v.at[j0 + k]],
                            add=True)

            @pl.when(jn + k < NCH)
            def _():
                pltpu.async_copy(tabx1.at[gidx_v.at[jn + k]], rows_v.at[k],
                                 sems[k])
        return carry

    lax.fori_loop(0, NCH // NBUF, grp, 0)
    plsc.subcore_barrier()
    pltpu.sync_copy(acc.at[pl.ds(base, RPW)], tpo.at[c, pl.ds(base, RPW)])


def _sc_p2d_body(tpf, cs, cd, gidx, sidx, zrows, uout, tabx, statsf,
                 gidx_v, sidx_v, rows_v, buf_v, t0_v, t1_v, cs_v, cd_v,
                 tb_v, sb_v, hl_v, tmp_v, fl_v, acc, hshs,
                 sem0, sem1, sem2, sem3):
    c = lax.axis_index("c")
    s = lax.axis_index("s")
    wid = s * NC + c
    base = s * RPW
    cnp = c * NP
    sems = (sem0, sem1, sem2, sem3)
    iota = lax.iota(jnp.int32, 16)
    rep = iota // B
    lane8 = iota % B

    pltpu.sync_copy(gidx.at[wid], gidx_v)
    pltpu.sync_copy(sidx.at[wid], sidx_v)
    pltpu.sync_copy(tpf.at[pl.ds(base * B, RPW * B)], t0_v)
    pltpu.sync_copy(tpf.at[pl.ds(NP * B + base * B, RPW * B)], t1_v)
    pltpu.sync_copy(cs.at[pl.ds(base, RPW)], cs_v)
    pltpu.sync_copy(cd.at[pl.ds(base, RPW)], cd_v)
    pltpu.sync_copy(zrows.at[pl.ds(base, RPW)], buf_v)
    pltpu.sync_copy(buf_v, acc.at[pl.ds(base, RPW)])

    def obody(j, carry):
        for k in range(CH // 16):
            gidx_v[j, pl.ds(k * 16, 16)] = (
                gidx_v[j, pl.ds(k * 16, 16)] + cnp)
        return carry

    lax.fori_loop(0, NCH, obody, 0)

    def tbody(i, carry):
        ps, qs, pm, qm = carry
        nidx = rep + 2 * i
        cdb = plsc.load_gather(cd_v, [nidx])
        csb = plsc.load_gather(cs_v, [nidx])
        t = t0_v[pl.ds(16 * i, 16)] + t1_v[pl.ds(16 * i, 16)]
        sv = t * cdb
        p = jnp.maximum(sv, 0.0)
        q = jnp.maximum(-sv, 0.0)
        plsc.store_scatter(tb_v, [nidx, lane8], p * csb)
        plsc.store_scatter(tb_v, [nidx, lane8 + B], q * csb)
        return (ps + p, qs + q, jnp.maximum(pm, p), jnp.maximum(qm, q))

    zv = jnp.zeros((16,), _f32)
    ps, qs, pm, qm = lax.fori_loop(0, RPW * B // 16, tbody,
                                   (zv, zv, zv, zv))
    pltpu.sync_copy(tb_v, tabx.at[pl.ds(cnp + base, RPW)])
    sb_v[pl.ds(0, 16)] = ps
    sb_v[pl.ds(16, 16)] = qs
    sb_v[pl.ds(32, 16)] = pm
    sb_v[pl.ds(48, 16)] = qm
    pltpu.sync_copy(sb_v, hshs.at[s])
    plsc.subcore_barrier()

    @pl.when(jnp.logical_and(c == 0, s == 0))
    def _():
        pltpu.sync_copy(hshs, hl_v)
        s0 = hl_v[0, pl.ds(0, 16)]
        s1 = hl_v[0, pl.ds(16, 16)]
        m0 = hl_v[0, pl.ds(32, 16)]
        m1 = hl_v[0, pl.ds(48, 16)]
        for r in range(1, NS):
            s0 = s0 + hl_v[r, pl.ds(0, 16)]
            s1 = s1 + hl_v[r, pl.ds(16, 16)]
            m0 = jnp.maximum(m0, hl_v[r, pl.ds(32, 16)])
            m1 = jnp.maximum(m1, hl_v[r, pl.ds(48, 16)])
        hi = jnp.minimum(iota + 8, 15)
        lo_mask = iota < 8

        def fold(v):
            tmp_v[...] = v
            return v + plsc.load_gather(tmp_v, [hi])

        inv_n = _f32(1.0 / N)
        plsc.store_scatter(fl_v, [iota], fold(s0) * inv_n,
                           mask=lo_mask)
        plsc.store_scatter(fl_v, [iota + 8], fold(s1) * inv_n,
                           mask=lo_mask)
        plsc.store_scatter(fl_v, [iota + 16], fold_max(m0, tmp_v, hi),
                           mask=lo_mask)
        plsc.store_scatter(fl_v, [iota + 24], fold_max(m1, tmp_v, hi),
                           mask=lo_mask)
        pltpu.sync_copy(fl_v, statsf)

    for k in range(NBUF):
        pltpu.async_copy(tabx.at[gidx_v.at[k]], rows_v.at[k], sems[k])

    def grp(g, carry):
        j0 = g * NBUF
        jn = j0 + NBUF
        for k in range(NBUF):
            pltpu.make_async_copy(tabx.at[gidx_v.at[j0 + k]],
                                  rows_v.at[k], sems[k]).wait()
            pltpu.sync_copy(rows_v.at[k], acc.at[sidx_v.at[j0 + k]],
                            add=True)

            @pl.when(jn + k < NCH)
            def _():
                pltpu.async_copy(tabx.at[gidx_v.at[jn + k]], rows_v.at[k],
                                 sems[k])
        return carry

    lax.fori_loop(0, NCH // NBUF, grp, 0)
    plsc.subcore_barrier()
    pltpu.sync_copy(acc.at[pl.ds(base, RPW)], uout.at[c, pl.ds(base, RPW)])


def fold_max(v, tmp_v, hi):
    tmp_v[...] = v
    return jnp.maximum(v, plsc.load_gather(tmp_v, [hi]))


_SC_PASS_CACHE = {}


def _sc_mesh():
    return plsc.VectorSubcoreMesh(core_axis_name="c", subcore_axis_name="s",
                                  num_cores=NC, num_subcores=NS)


def _sc_pass(table, gidx, sidx, zrows):
    w = table.shape[-1]
    fn = _SC_PASS_CACHE.get(("pass", w))
    if fn is None:
        fn = pl.kernel(
            _sc_pass_body,
            out_type=jax.ShapeDtypeStruct((NC, NP, w), _f32),
            mesh=_sc_mesh(),
            scratch_types=[
                pltpu.VMEM((NCH, CH), jnp.int32),
                pltpu.VMEM((NCH, CH), jnp.int32),
                pltpu.VMEM((NBUF, CH, w), _f32),
                pltpu.VMEM((RPW, w), _f32),
                pltpu.VMEM_SHARED((NP, w), _f32),
                pltpu.SemaphoreType.DMA,
                pltpu.SemaphoreType.DMA,
                pltpu.SemaphoreType.DMA,
                pltpu.SemaphoreType.DMA,
            ],
            compiler_params=pltpu.CompilerParams(use_tc_tiling_on_sc=False),
        )
        _SC_PASS_CACHE[("pass", w)] = fn
    return fn(table, gidx, sidx, zrows)


def _sc_p2d(*args):
    fn = _SC_PASS_CACHE.get("p2d")
    if fn is None:
        fn = pl.kernel(
            _sc_p2d_body,
            out_type=[jax.ShapeDtypeStruct((NC, NP, 16), _f32),
                      jax.ShapeDtypeStruct((NC * NP, 16), _f32),
                      jax.ShapeDtypeStruct((32,), _f32)],
            mesh=_sc_mesh(),
            scratch_types=[
                pltpu.VMEM((NCH, CH), jnp.int32),
                pltpu.VMEM((NCH, CH), jnp.int32),
                pltpu.VMEM((NBUF, CH, 16), _f32),
                pltpu.VMEM((RPW, 16), _f32),
                pltpu.VMEM((RPW * B,), _f32),
                pltpu.VMEM((RPW * B,), _f32),
                pltpu.VMEM((RPW,), _f32),
                pltpu.VMEM((RPW,), _f32),
                pltpu.VMEM((RPW, 16), _f32),
                pltpu.VMEM((64,), _f32),
                pltpu.VMEM((NS, 64), _f32),
                pltpu.VMEM((16,), _f32),
                pltpu.VMEM((32,), _f32),
                pltpu.VMEM_SHARED((NP, 16), _f32),
                pltpu.VMEM_SHARED((NS, 64), _f32),
                pltpu.SemaphoreType.DMA,
                pltpu.SemaphoreType.DMA,
                pltpu.SemaphoreType.DMA,
                pltpu.SemaphoreType.DMA,
            ],
            compiler_params=pltpu.CompilerParams(use_tc_tiling_on_sc=False,
                                                 needs_layout_passes=False),
        )
        _SC_PASS_CACHE["p2d"] = fn
    return fn(*args)


def _sc_deg1(*args):
    fn = _SC_PASS_CACHE.get("deg1")
    if fn is None:
        fn = pl.kernel(
            _sc_deg1_body,
            out_type=[jax.ShapeDtypeStruct((NC, NP, B), _f32),
                      jax.ShapeDtypeStruct((NC * NP, B), _f32),
                      jax.ShapeDtypeStruct((NP,), _f32),
                      jax.ShapeDtypeStruct((NP,), _f32)],
            mesh=_sc_mesh(),
            scratch_types=[
                pltpu.VMEM((2, NCH, CH), jnp.int32),
                pltpu.VMEM((2, NCH, CH), jnp.int32),
                pltpu.VMEM((NCH, CH), jnp.int32),
                pltpu.VMEM((NCH, CH), jnp.int32),
                pltpu.VMEM((NBUF, CH, B), _f32),
                pltpu.VMEM((RPW, B), _f32),
                pltpu.VMEM((RPW * B,), _f32),
                pltpu.VMEM((RPW, B), _f32),
                pltpu.VMEM((NP,), _f32),
                pltpu.VMEM((NP,), _f32),
                pltpu.VMEM((NS, RPW), _f32),
                pltpu.VMEM((RPW,), _f32),
                pltpu.VMEM((RPW,), _f32),
                pltpu.VMEM_SHARED((NP, B), _f32),
                pltpu.VMEM_SHARED((NS, NP), _f32),
                pltpu.VMEM_SHARED((NS, NP), _f32),
                pltpu.SemaphoreType.DMA,
                pltpu.SemaphoreType.DMA,
                pltpu.SemaphoreType.DMA,
                pltpu.SemaphoreType.DMA,
            ],
            compiler_params=pltpu.CompilerParams(use_tc_tiling_on_sc=False,
                                                 needs_layout_passes=False),
        )
        _SC_PASS_CACHE["deg1"] = fn
    return fn(*args)


def _f_body(up_ref, cd1_ref, stats_ref, w1_ref, w2_ref, b2_ref, inpc_ref,
            we_ref, h1w_ref, h1b_ref, h2w_ref, h2b_ref, h3w_ref, out_ref):
    up = up_ref[...]
    u_agg = up[0] + up[1]
    cd8 = jnp.broadcast_to(cd1_ref[...], (NP, B))
    pp = u_agg[:, 0:8] * cd8
    qp = u_agg[:, 8:16] * cd8
    w1 = w1_ref[...]
    w1p = jnp.maximum(w1, 0.0)
    w1n = jnp.maximum(-w1, 0.0)
    w2 = w2_ref[...]
    u = jnp.dot(w1p, w2, preferred_element_type=_f32)
    v = jnp.dot(w1n, w2, preferred_element_type=_f32)
    b2 = b2_ref[...]
    pq = jnp.concatenate([pp, qp], axis=1)
    colrep = lax.broadcasted_iota(jnp.int32, (16, 256), 1) // 32
    row16 = lax.broadcasted_iota(jnp.int32, (16, 256), 0)
    u_t = jnp.concatenate([u] * B, axis=1)
    v_t = jnp.concatenate([v] * B, axis=1)
    zero16 = jnp.zeros((16, 256), _f32)
    m = (jnp.where(row16 == colrep, u_t + zero16, 0.0)
         + jnp.where(row16 - B == colrep, v_t + zero16, 0.0))
    b2_t = jnp.concatenate([b2] * B, axis=1)
    x2 = jnp.maximum(
        jnp.dot(pq, m, preferred_element_type=_f32) + b2_t, 0.0)
    valid = lax.broadcasted_iota(jnp.int32, (NP, 256), 0) < N
    neg = _f32(-3.0e38)
    sums = jnp.sum(jnp.where(valid, x2, 0.0), axis=0, keepdims=True)
    maxs = jnp.max(jnp.where(valid, x2, neg), axis=0, keepdims=True)
    mean2 = jnp.concatenate(
        [sums[:, 32 * b:32 * (b + 1)] for b in range(B)],
        axis=0) * _f32(1.0 / N)
    max2 = jnp.concatenate(
        [maxs[:, 32 * b:32 * (b + 1)] for b in range(B)], axis=0)
    r2 = jnp.maximum(jnp.concatenate([mean2, max2], axis=1), 0.0)

    st = stats_ref[...]
    dn = (((0,), (0,)), ((), ()))
    mean1 = (lax.dot_general(st[0:1], w1p, dn, preferred_element_type=_f32)
             + lax.dot_general(st[1:2], w1n, dn, preferred_element_type=_f32))
    max1 = jnp.maximum(
        lax.dot_general(st[2:3], w1p, dn, preferred_element_type=_f32),
        lax.dot_general(st[3:4], w1n, dn, preferred_element_type=_f32))
    r1 = jnp.maximum(jnp.concatenate([mean1, max1], axis=1), 0.0)

    hg = r1 + r2
    embed = jnp.maximum(
        jnp.dot(inpc_ref[...], we_ref[...], preferred_element_type=_f32), 0.0)
    fusion = jnp.concatenate([embed, hg], axis=1)
    h = jnp.maximum(
        jnp.dot(fusion, h1w_ref[...], preferred_element_type=_f32)
        + h1b_ref[...], 0.0)
    h = jnp.maximum(
        jnp.dot(h, h2w_ref[...], preferred_element_type=_f32)
        + h2b_ref[...], 0.0)
    out_ref[...] = jnp.dot(h, h3w_ref[...], preferred_element_type=_f32)


def kernel(inp, edge_index, W1, b1, W2, b2, We, H1w, H1b, H2w, H2b, H3w):
    src = edge_index[0]
    dst = edge_index[1]
    pad = EP - E
    srcp = jnp.concatenate(
        [src, jnp.full((pad,), N, jnp.int32)]).reshape(NW, NCH, CH)
    dstp = jnp.concatenate(
        [dst, jnp.full((pad,), N, jnp.int32)]).reshape(NW, NCH, CH)
    x0t = jnp.pad(inp[:, CIN:].T, ((0, NP - N), (0, 0)))
    zrows8 = jnp.zeros((NP, 8), _f32)
    zrows16 = jnp.zeros((NP, 16), _f32)

    tp, _, cs, cd = _sc_deg1(srcp, dstp, x0t.reshape(-1), zrows8)
    cd1 = cd.reshape(NP, 1)

    up, _, statsf = _sc_p2d(tp.reshape(-1), cs, cd, srcp, dstp, zrows16)
    stats = statsf.reshape(4, B)

    out = pl.pallas_call(
        _f_body,
        out_shape=jax.ShapeDtypeStruct((B, 1), _f32),
    )(up, cd1, stats, W1, W2, b2.reshape(1, 32), inp[:, :CIN], We,
      H1w, H1b.reshape(1, 128), H2w, H2b.reshape(1, 64), H3w)
    return out


# speedup vs baseline: 1.1323x; 1.1323x over previous
import jax
import jax.numpy as jnp
from jax import lax
from jax.experimental import pallas as pl
from jax.experimental.pallas import tpu as pltpu
from jax.experimental.pallas import tpu_sc as plsc

N = 10000
NP = 10240
E = 160000
EP = 163840
CIN = 32
B = 8
NW = 32
NCH = 40
CH = 128
NC = 2
NS = 16
RPW = NP // NS

_f32 = jnp.float32


NBUF = 4


def _sc_pass_body(table, gidx, sidx, zrows, out, gidx_v, sidx_v, rows_v,
                  buf_v, acc, sem0, sem1, sem2, sem3):
    c = lax.axis_index("c")
    s = lax.axis_index("s")
    wid = s * NC + c
    base = s * RPW
    sems = (sem0, sem1, sem2, sem3)
    pltpu.sync_copy(gidx.at[wid], gidx_v)
    for k in range(NBUF):
        pltpu.async_copy(table.at[gidx_v.at[k]], rows_v.at[k], sems[k])
    pltpu.sync_copy(sidx.at[wid], sidx_v)
    pltpu.sync_copy(zrows.at[pl.ds(base, RPW)], buf_v)
    pltpu.sync_copy(buf_v, acc.at[pl.ds(base, RPW)])
    plsc.subcore_barrier()

    def grp(g, carry):
        j0 = g * NBUF
        jn = j0 + NBUF
        for k in range(NBUF):
            pltpu.make_async_copy(table.at[gidx_v.at[j0 + k]],
                                  rows_v.at[k], sems[k]).wait()
            pltpu.sync_copy(rows_v.at[k], acc.at[sidx_v.at[j0 + k]],
                            add=True)

            @pl.when(jn + k < NCH)
            def _():
                pltpu.async_copy(table.at[gidx_v.at[jn + k]], rows_v.at[k],
                                 sems[k])
        return carry

    lax.fori_loop(0, NCH // NBUF, grp, 0)
    plsc.subcore_barrier()
    pltpu.sync_copy(acc.at[pl.ds(base, RPW)], out.at[c, pl.ds(base, RPW)])


NPT = EP // NW // 16


def _sc_degb_body(sidx, didx, x0f, tab1, cso, cdo,
                  idx_v, x0_v, tb_v, hist_v, red_v, cs_v, hsh):
    c = lax.axis_index("c")
    s = lax.axis_index("s")
    base = s * RPW

    @pl.when(c == 0)
    def _():
        pltpu.sync_copy(sidx.at[pl.ds(2 * s, 2)], idx_v)

    @pl.when(c != 0)
    def _():
        pltpu.sync_copy(didx.at[pl.ds(2 * s, 2)], idx_v)

    def zbody(i, carry):
        hist_v[pl.ds(i * 16, 16)] = jnp.zeros((16,), _f32)
        return carry

    lax.fori_loop(0, NP // 16, zbody, 0)
    ones = jnp.full((16,), 1.0, _f32)

    def hbody(i, carry):
        plsc.addupdate_scatter(hist_v, [idx_v[0, i, :]], ones)
        plsc.addupdate_scatter(hist_v, [idx_v[1, i, :]], ones)
        return carry

    lax.fori_loop(0, NPT, hbody, 0)
    pltpu.sync_copy(hist_v, hsh.at[s])
    plsc.subcore_barrier()
    for r in range(NS):
        pltpu.sync_copy(hsh.at[r, pl.ds(base, RPW)], red_v.at[r])

    def cbody(i, carry):
        tot = red_v[0, pl.ds(i * 16, 16)]
        for r in range(1, NS):
            tot = tot + red_v[r, pl.ds(i * 16, 16)]
        x = jnp.maximum(tot, 1.0)
        yi = 0x5F3759DF - lax.shift_right_logical(
            plsc.bitcast(x, jnp.int32), 1)
        y = plsc.bitcast(yi, _f32)
        for _ in range(3):
            y = y * (1.5 - 0.5 * x * y * y)
        cs_v[pl.ds(i * 16, 16)] = y
        return carry

    lax.fori_loop(0, RPW // 16, cbody, 0)

    @pl.when(c == 0)
    def _():
        pltpu.sync_copy(cs_v, cso.at[pl.ds(base, RPW)])
        pltpu.sync_copy(x0f.at[pl.ds(base * B, RPW * B)], x0_v)
        rep = lax.iota(jnp.int32, 16) // B

        def tbody(i, carry):
            csb = plsc.load_gather(cs_v, [rep + 2 * i])
            tb_v[pl.ds(i * 16, 16)] = x0_v[pl.ds(i * 16, 16)] * csb
            return carry

        lax.fori_loop(0, RPW * B // 16, tbody, 0)
        pltpu.sync_copy(tb_v, tab1.at[pl.ds(base * B, RPW * B)])

    @pl.when(c != 0)
    def _():
        pltpu.sync_copy(cs_v, cdo.at[pl.ds(base, RPW)])


def _sc_p2d_body(tpf, cs, cd, gidx, sidx, zrows, uout, tabx, statsf,
                 gidx_v, sidx_v, rows_v, buf_v, t0_v, t1_v, cs_v, cd_v,
                 tb_v, sb_v, hl_v, tmp_v, fl_v, acc, hshs,
                 sem0, sem1, sem2, sem3):
    c = lax.axis_index("c")
    s = lax.axis_index("s")
    wid = s * NC + c
    base = s * RPW
    cnp = c * NP
    sems = (sem0, sem1, sem2, sem3)
    iota = lax.iota(jnp.int32, 16)
    rep = iota // B
    lane8 = iota % B

    pltpu.sync_copy(gidx.at[wid], gidx_v)
    pltpu.sync_copy(sidx.at[wid], sidx_v)
    pltpu.sync_copy(tpf.at[pl.ds(base * B, RPW * B)], t0_v)
    pltpu.sync_copy(tpf.at[pl.ds(NP * B + base * B, RPW * B)], t1_v)
    pltpu.sync_copy(cs.at[pl.ds(base, RPW)], cs_v)
    pltpu.sync_copy(cd.at[pl.ds(base, RPW)], cd_v)
    pltpu.sync_copy(zrows.at[pl.ds(base, RPW)], buf_v)
    pltpu.sync_copy(buf_v, acc.at[pl.ds(base, RPW)])

    def obody(j, carry):
        for k in range(CH // 16):
            gidx_v[j, pl.ds(k * 16, 16)] = (
                gidx_v[j, pl.ds(k * 16, 16)] + cnp)
        return carry

    lax.fori_loop(0, NCH, obody, 0)

    def tbody(i, carry):
        ps, qs, pm, qm = carry
        nidx = rep + 2 * i
        cdb = plsc.load_gather(cd_v, [nidx])
        csb = plsc.load_gather(cs_v, [nidx])
        t = t0_v[pl.ds(16 * i, 16)] + t1_v[pl.ds(16 * i, 16)]
        sv = t * cdb
        p = jnp.maximum(sv, 0.0)
        q = jnp.maximum(-sv, 0.0)
        plsc.store_scatter(tb_v, [nidx, lane8], p * csb)
        plsc.store_scatter(tb_v, [nidx, lane8 + B], q * csb)
        return (ps + p, qs + q, jnp.maximum(pm, p), jnp.maximum(qm, q))

    zv = jnp.zeros((16,), _f32)
    ps, qs, pm, qm = lax.fori_loop(0, RPW * B // 16, tbody,
                                   (zv, zv, zv, zv))
    pltpu.sync_copy(tb_v, tabx.at[pl.ds(cnp + base, RPW)])
    sb_v[pl.ds(0, 16)] = ps
    sb_v[pl.ds(16, 16)] = qs
    sb_v[pl.ds(32, 16)] = pm
    sb_v[pl.ds(48, 16)] = qm
    pltpu.sync_copy(sb_v, hshs.at[s])
    plsc.subcore_barrier()

    @pl.when(jnp.logical_and(c == 0, s == 0))
    def _():
        pltpu.sync_copy(hshs, hl_v)
        s0 = hl_v[0, pl.ds(0, 16)]
        s1 = hl_v[0, pl.ds(16, 16)]
        m0 = hl_v[0, pl.ds(32, 16)]
        m1 = hl_v[0, pl.ds(48, 16)]
        for r in range(1, NS):
            s0 = s0 + hl_v[r, pl.ds(0, 16)]
            s1 = s1 + hl_v[r, pl.ds(16, 16)]
            m0 = jnp.maximum(m0, hl_v[r, pl.ds(32, 16)])
            m1 = jnp.maximum(m1, hl_v[r, pl.ds(48, 16)])
        hi = jnp.minimum(iota + 8, 15)
        lo_mask = iota < 8

        def fold(v):
            tmp_v[...] = v
            return v + plsc.load_gather(tmp_v, [hi])

        inv_n = _f32(1.0 / N)
        plsc.store_scatter(fl_v, [iota], fold(s0) * inv_n,
                           mask=lo_mask)
        plsc.store_scatter(fl_v, [iota + 8], fold(s1) * inv_n,
                           mask=lo_mask)
        plsc.store_scatter(fl_v, [iota + 16], fold_max(m0, tmp_v, hi),
                           mask=lo_mask)
        plsc.store_scatter(fl_v, [iota + 24], fold_max(m1, tmp_v, hi),
                           mask=lo_mask)
        pltpu.sync_copy(fl_v, statsf)

    for k in range(NBUF):
        pltpu.async_copy(tabx.at[gidx_v.at[k]], rows_v.at[k], sems[k])

    def grp(g, carry):
        j0 = g * NBUF
        jn = j0 + NBUF
        for k in range(NBUF):
            pltpu.make_async_copy(tabx.at[gidx_v.at[j0 + k]],
                                  rows_v.at[k], sems[k]).wait()
            pltpu.sync_copy(rows_v.at[k], acc.at[sidx_v.at[j0 + k]],
                            add=True)

            @pl.when(jn + k < NCH)
            def _():
                pltpu.async_copy(tabx.at[gidx_v.at[jn + k]], rows_v.at[k],
                                 sems[k])
        return carry

    lax.fori_loop(0, NCH // NBUF, grp, 0)
    plsc.subcore_barrier()
    pltpu.sync_copy(acc.at[pl.ds(base, RPW)], uout.at[c, pl.ds(base, RPW)])


def fold_max(v, tmp_v, hi):
    tmp_v[...] = v
    return jnp.maximum(v, plsc.load_gather(tmp_v, [hi]))


_SC_PASS_CACHE = {}


def _sc_mesh():
    return plsc.VectorSubcoreMesh(core_axis_name="c", subcore_axis_name="s",
                                  num_cores=NC, num_subcores=NS)


def _sc_pass(table, gidx, sidx, zrows):
    w = table.shape[-1]
    fn = _SC_PASS_CACHE.get(("pass", w))
    if fn is None:
        fn = pl.kernel(
            _sc_pass_body,
            out_type=jax.ShapeDtypeStruct((NC, NP, w), _f32),
            mesh=_sc_mesh(),
            scratch_types=[
                pltpu.VMEM((NCH, CH), jnp.int32),
                pltpu.VMEM((NCH, CH), jnp.int32),
                pltpu.VMEM((NBUF, CH, w), _f32),
                pltpu.VMEM((RPW, w), _f32),
                pltpu.VMEM_SHARED((NP, w), _f32),
                pltpu.SemaphoreType.DMA,
                pltpu.SemaphoreType.DMA,
                pltpu.SemaphoreType.DMA,
                pltpu.SemaphoreType.DMA,
            ],
            compiler_params=pltpu.CompilerParams(use_tc_tiling_on_sc=False),
        )
        _SC_PASS_CACHE[("pass", w)] = fn
    return fn(table, gidx, sidx, zrows)


def _sc_p2d(*args):
    fn = _SC_PASS_CACHE.get("p2d")
    if fn is None:
        fn = pl.kernel(
            _sc_p2d_body,
            out_type=[jax.ShapeDtypeStruct((NC, NP, 16), _f32),
                      jax.ShapeDtypeStruct((NC * NP, 16), _f32),
                      jax.ShapeDtypeStruct((32,), _f32)],
            mesh=_sc_mesh(),
            scratch_types=[
                pltpu.VMEM((NCH, CH), jnp.int32),
                pltpu.VMEM((NCH, CH), jnp.int32),
                pltpu.VMEM((NBUF, CH, 16), _f32),
                pltpu.VMEM((RPW, 16), _f32),
                pltpu.VMEM((RPW * B,), _f32),
                pltpu.VMEM((RPW * B,), _f32),
                pltpu.VMEM((RPW,), _f32),
                pltpu.VMEM((RPW,), _f32),
                pltpu.VMEM((RPW, 16), _f32),
                pltpu.VMEM((64,), _f32),
                pltpu.VMEM((NS, 64), _f32),
                pltpu.VMEM((16,), _f32),
                pltpu.VMEM((32,), _f32),
                pltpu.VMEM_SHARED((NP, 16), _f32),
                pltpu.VMEM_SHARED((NS, 64), _f32),
                pltpu.SemaphoreType.DMA,
                pltpu.SemaphoreType.DMA,
                pltpu.SemaphoreType.DMA,
                pltpu.SemaphoreType.DMA,
            ],
            compiler_params=pltpu.CompilerParams(use_tc_tiling_on_sc=False,
                                                 needs_layout_passes=False),
        )
        _SC_PASS_CACHE["p2d"] = fn
    return fn(*args)


def _sc_degb(*args):
    fn = _SC_PASS_CACHE.get("degb")
    if fn is None:
        fn = pl.kernel(
            _sc_degb_body,
            out_type=[jax.ShapeDtypeStruct((NP * B,), _f32),
                      jax.ShapeDtypeStruct((NP,), _f32),
                      jax.ShapeDtypeStruct((NP,), _f32)],
            mesh=_sc_mesh(),
            scratch_types=[
                pltpu.VMEM((2, NPT, 16), jnp.int32),
                pltpu.VMEM((RPW * B,), _f32),
                pltpu.VMEM((RPW * B,), _f32),
                pltpu.VMEM((NP,), _f32),
                pltpu.VMEM((NS, RPW), _f32),
                pltpu.VMEM((RPW,), _f32),
                pltpu.VMEM_SHARED((NS, NP), _f32),
            ],
            compiler_params=pltpu.CompilerParams(use_tc_tiling_on_sc=False,
                                                 needs_layout_passes=False),
        )
        _SC_PASS_CACHE["degb"] = fn
    return fn(*args)


def _f_body(up_ref, cd1_ref, stats_ref, w1_ref, w2_ref, b2_ref, inpc_ref,
            we_ref, h1w_ref, h1b_ref, h2w_ref, h2b_ref, h3w_ref, out_ref):
    up = up_ref[...]
    u_agg = up[0] + up[1]
    cd8 = jnp.broadcast_to(cd1_ref[...], (NP, B))
    pp = u_agg[:, 0:8] * cd8
    qp = u_agg[:, 8:16] * cd8
    w1 = w1_ref[...]
    w1p = jnp.maximum(w1, 0.0)
    w1n = jnp.maximum(-w1, 0.0)
    w2 = w2_ref[...]
    u = jnp.dot(w1p, w2, preferred_element_type=_f32)
    v = jnp.dot(w1n, w2, preferred_element_type=_f32)
    b2 = b2_ref[...]
    pq = jnp.concatenate([pp, qp], axis=1)
    colrep = lax.broadcasted_iota(jnp.int32, (16, 256), 1) // 32
    row16 = lax.broadcasted_iota(jnp.int32, (16, 256), 0)
    u_t = jnp.concatenate([u] * B, axis=1)
    v_t = jnp.concatenate([v] * B, axis=1)
    zero16 = jnp.zeros((16, 256), _f32)
    m = (jnp.where(row16 == colrep, u_t + zero16, 0.0)
         + jnp.where(row16 - B == colrep, v_t + zero16, 0.0))
    b2_t = jnp.concatenate([b2] * B, axis=1)
    x2 = jnp.maximum(
        jnp.dot(pq, m, preferred_element_type=_f32) + b2_t, 0.0)
    valid = lax.broadcasted_iota(jnp.int32, (NP, 256), 0) < N
    neg = _f32(-3.0e38)
    sums = jnp.sum(jnp.where(valid, x2, 0.0), axis=0, keepdims=True)
    maxs = jnp.max(jnp.where(valid, x2, neg), axis=0, keepdims=True)
    mean2 = jnp.concatenate(
        [sums[:, 32 * b:32 * (b + 1)] for b in range(B)],
        axis=0) * _f32(1.0 / N)
    max2 = jnp.concatenate(
        [maxs[:, 32 * b:32 * (b + 1)] for b in range(B)], axis=0)
    r2 = jnp.maximum(jnp.concatenate([mean2, max2], axis=1), 0.0)

    st = stats_ref[...]
    dn = (((0,), (0,)), ((), ()))
    mean1 = (lax.dot_general(st[0:1], w1p, dn, preferred_element_type=_f32)
             + lax.dot_general(st[1:2], w1n, dn, preferred_element_type=_f32))
    max1 = jnp.maximum(
        lax.dot_general(st[2:3], w1p, dn, preferred_element_type=_f32),
        lax.dot_general(st[3:4], w1n, dn, preferred_element_type=_f32))
    r1 = jnp.maximum(jnp.concatenate([mean1, max1], axis=1), 0.0)

    hg = r1 + r2
    embed = jnp.maximum(
        jnp.dot(inpc_ref[...], we_ref[...], preferred_element_type=_f32), 0.0)
    fusion = jnp.concatenate([embed, hg], axis=1)
    h = jnp.maximum(
        jnp.dot(fusion, h1w_ref[...], preferred_element_type=_f32)
        + h1b_ref[...], 0.0)
    h = jnp.maximum(
        jnp.dot(h, h2w_ref[...], preferred_element_type=_f32)
        + h2b_ref[...], 0.0)
    out_ref[...] = jnp.dot(h, h3w_ref[...], preferred_element_type=_f32)


def kernel(inp, edge_index, W1, b1, W2, b2, We, H1w, H1b, H2w, H2b, H3w):
    src = edge_index[0]
    dst = edge_index[1]
    pad = EP - E
    srcp = jnp.concatenate(
        [src, jnp.full((pad,), N, jnp.int32)]).reshape(NW, NCH, CH)
    dstp = jnp.concatenate(
        [dst, jnp.full((pad,), N, jnp.int32)]).reshape(NW, NCH, CH)
    x0t = jnp.pad(inp[:, CIN:].T, ((0, NP - N), (0, 0)))
    zrows8 = jnp.zeros((NP, 8), _f32)
    zrows16 = jnp.zeros((NP, 16), _f32)

    tab1f, cs, cd = _sc_degb(srcp.reshape(NW, NPT, 16),
                             dstp.reshape(NW, NPT, 16), x0t.reshape(-1))
    tab1 = tab1f.reshape(NP, B)
    cd1 = cd.reshape(NP, 1)

    tp = _sc_pass(tab1, srcp, dstp, zrows8)

    up, _, statsf = _sc_p2d(tp.reshape(-1), cs, cd, srcp, dstp, zrows16)
    stats = statsf.reshape(4, B)

    out = pl.pallas_call(
        _f_body,
        out_shape=jax.ShapeDtypeStruct((B, 1), _f32),
    )(up, cd1, stats, W1, W2, b2.reshape(1, 32), inp[:, :CIN], We,
      H1w, H1b.reshape(1, 128), H2w, H2b.reshape(1, 64), H3w)
    return out
